# baseline (speedup 1.0000x reference)
import functools

import jax
import jax.numpy as jnp
from jax.experimental import pallas as pl
from jax.experimental.pallas import tpu as pltpu

_BN_EPS = 1e-5


def _conv_stats_kernel(x_ref, w1_ref, y_ref, st_ref, xw_ref, xt_ref, *, H, W,
                       Cin, C1):
    HW = H * W
    x = x_ref[0]
    for kx in range(3):
        xw_ref[:, :, kx * Cin:(kx + 1) * Cin] = (
            x[:, kx:kx + W, :].astype(jnp.bfloat16))
    for ky in range(3):
        xt_ref[:, ky * 3 * Cin:(ky + 1) * 3 * Cin] = (
            xw_ref[ky:ky + H].reshape(HW, 3 * Cin))
    acc = jnp.dot(xt_ref[...], w1_ref[...],
                  preferred_element_type=jnp.float32)
    KQ = HW // 4
    y4 = jnp.concatenate([acc[q * KQ:(q + 1) * KQ] for q in range(4)],
                         axis=1)
    y_ref[0] = y4
    st_ref[0, 0:1, :] = jnp.sum(y4, axis=0, keepdims=True)
    st_ref[0, 1:2, :] = jnp.sum(y4 * y4, axis=0, keepdims=True)


def _head_kernel(y_ref, ss_ref, w2_ref, b2_ref, out_ref, *, C1, O):
    y4 = y_ref[0]
    z4 = jnp.maximum(y4 * ss_ref[0:1, :] + ss_ref[1:2, :], 0.0)
    z4 = z4.astype(jnp.bfloat16)
    out4 = jnp.dot(z4, w2_ref[...], preferred_element_type=jnp.float32)
    out4 = out4 + b2_ref[...]
    KQ = y4.shape[0]
    for q in range(4):
        out_ref[0, q * KQ:(q + 1) * KQ, :] = out4[:, q * O:(q + 1) * O]


def kernel(x_nchw, w1, b1, gamma, beta, w2, b2):
    del b1
    N, Cin, H, W = x_nchw.shape
    C1 = w1.shape[-1]
    O = w2.shape[-1]
    HW = H * W
    rows = N * HW

    x_pad = jnp.pad(
        jnp.transpose(x_nchw, (0, 2, 3, 1)),
        ((0, 0), (1, 1), (1, 1), (0, 0)))
    w1b = w1.reshape(9 * Cin, C1).astype(jnp.bfloat16)
    w2m = w2.reshape(C1, O)
    w2blk = jnp.kron(jnp.eye(4, dtype=w2m.dtype), w2m).astype(jnp.bfloat16)
    b2t = jnp.tile(b2.reshape(1, O), (1, 4)).astype(jnp.float32)

    cparams = pltpu.CompilerParams(
        dimension_semantics=("arbitrary",),
        vmem_limit_bytes=64 * 1024 * 1024,
    )

    conv_flops = 2 * rows * 9 * Cin * C1
    y, stats = pl.pallas_call(
        functools.partial(_conv_stats_kernel, H=H, W=W, Cin=Cin, C1=C1),
        out_shape=(jax.ShapeDtypeStruct((N, HW // 4, 4 * C1), jnp.float32),
                   jax.ShapeDtypeStruct((N, 2, 4 * C1), jnp.float32)),
        grid=(N,),
        in_specs=[pl.BlockSpec((1, H + 2, W + 2, Cin), lambda n: (n, 0, 0, 0)),
                  pl.BlockSpec((9 * Cin, C1), lambda n: (0, 0))],
        out_specs=(pl.BlockSpec((1, HW // 4, 4 * C1), lambda n: (n, 0, 0)),
                   pl.BlockSpec((1, 2, 4 * C1), lambda n: (n, 0, 0))),
        scratch_shapes=[pltpu.VMEM((H + 2, W, 3 * Cin), jnp.bfloat16),
                        pltpu.VMEM((HW, 9 * Cin), jnp.bfloat16)],
        compiler_params=cparams,
        cost_estimate=pl.CostEstimate(
            flops=conv_flops, transcendentals=0,
            bytes_accessed=x_pad.size * 4 + w1b.size * 2
            + (rows + 8 * N) * C1 * 4),
    )(x_pad, w1b)

    s4 = jnp.sum(stats[:, 0, :], axis=0).reshape(4, C1)
    q4 = jnp.sum(stats[:, 1, :], axis=0).reshape(4, C1)
    mean = jnp.sum(s4, axis=0) * (1.0 / rows)
    var = jnp.maximum(jnp.sum(q4, axis=0) * (1.0 / rows) - mean * mean, 0.0)
    scale = gamma.reshape(C1) * jax.lax.rsqrt(var + _BN_EPS)
    shift = beta.reshape(C1) - mean * scale
    ss = jnp.stack([jnp.tile(scale, 4), jnp.tile(shift, 4)])

    out = pl.pallas_call(
        functools.partial(_head_kernel, C1=C1, O=O),
        out_shape=jax.ShapeDtypeStruct((N, HW, O), jnp.float32),
        grid=(N,),
        in_specs=[pl.BlockSpec((1, HW // 4, 4 * C1), lambda n: (n, 0, 0)),
                  pl.BlockSpec((2, 4 * C1), lambda n: (0, 0)),
                  pl.BlockSpec((4 * C1, 4 * O), lambda n: (0, 0)),
                  pl.BlockSpec((1, 4 * O), lambda n: (0, 0))],
        out_specs=pl.BlockSpec((1, HW, O), lambda n: (n, 0, 0)),
        compiler_params=cparams,
        cost_estimate=pl.CostEstimate(
            flops=2 * rows * C1 * O, transcendentals=0,
            bytes_accessed=rows * C1 * 4 + w2blk.size * 2 + rows * O * 4),
    )(y, ss, w2blk, b2t)

    out = out.reshape(N, H, W, O)
    return jnp.transpose(out, (0, 3, 1, 2))


# speedup vs baseline: 1.0999x; 1.0999x over previous
import functools

import jax
import jax.numpy as jnp
from jax.experimental import pallas as pl
from jax.experimental.pallas import tpu as pltpu

_BN_EPS = 1e-5


def _conv_stats_kernel(x_ref, w1_ref, y_ref, st_ref, xw_ref, *, H, W,
                       Cin, C1):
    HW = H * W
    KQ = HW // 4
    for i in range(x_ref.shape[0]):
        x = x_ref[i]
        for kx in range(3):
            xw_ref[:, :, kx * Cin:(kx + 1) * Cin] = (
                x[:, kx:kx + W, :].astype(jnp.bfloat16))
        acc = jnp.zeros((HW, C1), jnp.float32)
        for ky in range(3):
            acc = acc + jnp.dot(xw_ref[ky:ky + H].reshape(HW, 3 * Cin),
                                w1_ref[ky * 3 * Cin:(ky + 1) * 3 * Cin],
                                preferred_element_type=jnp.float32)
        y4 = jnp.concatenate([acc[q * KQ:(q + 1) * KQ] for q in range(4)],
                             axis=1)
        y_ref[i] = y4
        st_ref[i, 0:1, :] = jnp.sum(y4, axis=0, keepdims=True)
        st_ref[i, 1:2, :] = jnp.sum(y4 * y4, axis=0, keepdims=True)


def _head_kernel(y_ref, ss_ref, w2_ref, b2_ref, out_ref, *, C1, O):
    for i in range(y_ref.shape[0]):
        y4 = y_ref[i]
        z4 = jnp.maximum(y4 * ss_ref[0:1, :] + ss_ref[1:2, :], 0.0)
        z4 = z4.astype(jnp.bfloat16)
        out4 = jnp.dot(z4, w2_ref[...], preferred_element_type=jnp.float32)
        out4 = out4 + b2_ref[...]
        KQ = y4.shape[0]
        for q in range(4):
            out_ref[i, q * KQ:(q + 1) * KQ, :] = out4[:, q * O:(q + 1) * O]


def kernel(x_nchw, w1, b1, gamma, beta, w2, b2):
    del b1
    N, Cin, H, W = x_nchw.shape
    C1 = w1.shape[-1]
    O = w2.shape[-1]
    HW = H * W
    rows = N * HW

    x_pad = jnp.pad(
        jnp.transpose(x_nchw, (0, 2, 3, 1)),
        ((0, 0), (1, 1), (1, 1), (0, 0)))
    w1b = w1.reshape(9 * Cin, C1).astype(jnp.bfloat16)
    w2m = w2.reshape(C1, O)
    w2blk = jnp.kron(jnp.eye(4, dtype=w2m.dtype), w2m).astype(jnp.bfloat16)
    b2t = jnp.tile(b2.reshape(1, O), (1, 4)).astype(jnp.float32)

    cparams = pltpu.CompilerParams(
        dimension_semantics=("arbitrary",),
        vmem_limit_bytes=64 * 1024 * 1024,
    )

    conv_flops = 2 * rows * 9 * Cin * C1
    y, stats = pl.pallas_call(
        functools.partial(_conv_stats_kernel, H=H, W=W, Cin=Cin, C1=C1),
        out_shape=(jax.ShapeDtypeStruct((N, HW // 4, 4 * C1), jnp.float32),
                   jax.ShapeDtypeStruct((N, 2, 4 * C1), jnp.float32)),
        grid=(N // 2,),
        in_specs=[pl.BlockSpec((2, H + 2, W + 2, Cin), lambda n: (n, 0, 0, 0)),
                  pl.BlockSpec((9 * Cin, C1), lambda n: (0, 0))],
        out_specs=(pl.BlockSpec((2, HW // 4, 4 * C1), lambda n: (n, 0, 0)),
                   pl.BlockSpec((2, 2, 4 * C1), lambda n: (n, 0, 0))),
        scratch_shapes=[pltpu.VMEM((H + 2, W, 3 * Cin), jnp.bfloat16)],
        compiler_params=cparams,
        cost_estimate=pl.CostEstimate(
            flops=conv_flops, transcendentals=0,
            bytes_accessed=x_pad.size * 4 + w1b.size * 2
            + (rows + 8 * N) * C1 * 4),
    )(x_pad, w1b)

    s4 = jnp.sum(stats[:, 0, :], axis=0).reshape(4, C1)
    q4 = jnp.sum(stats[:, 1, :], axis=0).reshape(4, C1)
    mean = jnp.sum(s4, axis=0) * (1.0 / rows)
    var = jnp.maximum(jnp.sum(q4, axis=0) * (1.0 / rows) - mean * mean, 0.0)
    scale = gamma.reshape(C1) * jax.lax.rsqrt(var + _BN_EPS)
    shift = beta.reshape(C1) - mean * scale
    ss = jnp.stack([jnp.tile(scale, 4), jnp.tile(shift, 4)])

    out = pl.pallas_call(
        functools.partial(_head_kernel, C1=C1, O=O),
        out_shape=jax.ShapeDtypeStruct((N, HW, O), jnp.float32),
        grid=(N // 2,),
        in_specs=[pl.BlockSpec((2, HW // 4, 4 * C1), lambda n: (n, 0, 0)),
                  pl.BlockSpec((2, 4 * C1), lambda n: (0, 0)),
                  pl.BlockSpec((4 * C1, 4 * O), lambda n: (0, 0)),
                  pl.BlockSpec((1, 4 * O), lambda n: (0, 0))],
        out_specs=pl.BlockSpec((2, HW, O), lambda n: (n, 0, 0)),
        compiler_params=cparams,
        cost_estimate=pl.CostEstimate(
            flops=2 * rows * C1 * O, transcendentals=0,
            bytes_accessed=rows * C1 * 4 + w2blk.size * 2 + rows * O * 4),
    )(y, ss, w2blk, b2t)

    out = out.reshape(N, H, W, O)
    return jnp.transpose(out, (0, 3, 1, 2))


# speedup vs baseline: 1.1024x; 1.0023x over previous
import functools

import jax
import jax.numpy as jnp
from jax.experimental import pallas as pl
from jax.experimental.pallas import tpu as pltpu

_BN_EPS = 1e-5


def _conv_stats_kernel(x_ref, w1_ref, y_ref, st_ref, xw_ref, *, H, W,
                       Cin, C1):
    HW = H * W
    KQ = HW // 4
    for i in range(x_ref.shape[0]):
        x = x_ref[i]
        for kx in range(3):
            xw_ref[:, :, kx * Cin:(kx + 1) * Cin] = (
                x[:, kx:kx + W, :].astype(jnp.bfloat16))
        acc = jnp.zeros((HW, C1), jnp.float32)
        for ky in range(3):
            acc = acc + jnp.dot(xw_ref[ky:ky + H].reshape(HW, 3 * Cin),
                                w1_ref[ky * 3 * Cin:(ky + 1) * 3 * Cin],
                                preferred_element_type=jnp.float32)
        y4 = jnp.concatenate([acc[q * KQ:(q + 1) * KQ] for q in range(4)],
                             axis=1)
        y_ref[i] = y4
        st_ref[i, 0:1, :] = jnp.sum(y4, axis=0, keepdims=True)
        st_ref[i, 1:2, :] = jnp.sum(y4 * y4, axis=0, keepdims=True)


def _head_kernel(y_ref, ss_ref, w2_ref, b2_ref, out_ref, *, C1, O):
    for i in range(y_ref.shape[0]):
        y4 = y_ref[i]
        z4 = jnp.maximum(y4 * ss_ref[0:1, :] + ss_ref[1:2, :], 0.0)
        z4 = z4.astype(jnp.bfloat16)
        out4 = jnp.dot(z4, w2_ref[...], preferred_element_type=jnp.float32)
        out4 = out4 + b2_ref[...]
        KQ = y4.shape[0]
        for q in range(4):
            out_ref[i, q * KQ:(q + 1) * KQ, :] = out4[:, q * O:(q + 1) * O]


def kernel(x_nchw, w1, b1, gamma, beta, w2, b2):
    del b1
    N, Cin, H, W = x_nchw.shape
    C1 = w1.shape[-1]
    O = w2.shape[-1]
    HW = H * W
    rows = N * HW

    x_pad = jnp.pad(
        jnp.transpose(x_nchw, (0, 2, 3, 1)),
        ((0, 0), (1, 1), (1, 1), (0, 0)))
    w1b = w1.reshape(9 * Cin, C1).astype(jnp.bfloat16)
    w2m = w2.reshape(C1, O)
    w2blk = jnp.kron(jnp.eye(4, dtype=w2m.dtype), w2m).astype(jnp.bfloat16)
    b2t = jnp.tile(b2.reshape(1, O), (1, 4)).astype(jnp.float32)

    cparams = pltpu.CompilerParams(
        dimension_semantics=("arbitrary",),
        vmem_limit_bytes=64 * 1024 * 1024,
    )

    conv_flops = 2 * rows * 9 * Cin * C1
    y, stats = pl.pallas_call(
        functools.partial(_conv_stats_kernel, H=H, W=W, Cin=Cin, C1=C1),
        out_shape=(jax.ShapeDtypeStruct((N, HW // 4, 4 * C1), jnp.float32),
                   jax.ShapeDtypeStruct((N, 2, 4 * C1), jnp.float32)),
        grid=(N // 4,),
        in_specs=[pl.BlockSpec((4, H + 2, W + 2, Cin), lambda n: (n, 0, 0, 0)),
                  pl.BlockSpec((9 * Cin, C1), lambda n: (0, 0))],
        out_specs=(pl.BlockSpec((4, HW // 4, 4 * C1), lambda n: (n, 0, 0)),
                   pl.BlockSpec((4, 2, 4 * C1), lambda n: (n, 0, 0))),
        scratch_shapes=[pltpu.VMEM((H + 2, W, 3 * Cin), jnp.bfloat16)],
        compiler_params=cparams,
        cost_estimate=pl.CostEstimate(
            flops=conv_flops, transcendentals=0,
            bytes_accessed=x_pad.size * 4 + w1b.size * 2
            + (rows + 8 * N) * C1 * 4),
    )(x_pad, w1b)

    s4 = jnp.sum(stats[:, 0, :], axis=0).reshape(4, C1)
    q4 = jnp.sum(stats[:, 1, :], axis=0).reshape(4, C1)
    mean = jnp.sum(s4, axis=0) * (1.0 / rows)
    var = jnp.maximum(jnp.sum(q4, axis=0) * (1.0 / rows) - mean * mean, 0.0)
    scale = gamma.reshape(C1) * jax.lax.rsqrt(var + _BN_EPS)
    shift = beta.reshape(C1) - mean * scale
    ss = jnp.stack([jnp.tile(scale, 4), jnp.tile(shift, 4)])

    out = pl.pallas_call(
        functools.partial(_head_kernel, C1=C1, O=O),
        out_shape=jax.ShapeDtypeStruct((N, HW, O), jnp.float32),
        grid=(N // 4,),
        in_specs=[pl.BlockSpec((4, HW // 4, 4 * C1), lambda n: (n, 0, 0)),
                  pl.BlockSpec((2, 4 * C1), lambda n: (0, 0)),
                  pl.BlockSpec((4 * C1, 4 * O), lambda n: (0, 0)),
                  pl.BlockSpec((1, 4 * O), lambda n: (0, 0))],
        out_specs=pl.BlockSpec((4, HW, O), lambda n: (n, 0, 0)),
        compiler_params=cparams,
        cost_estimate=pl.CostEstimate(
            flops=2 * rows * C1 * O, transcendentals=0,
            bytes_accessed=rows * C1 * 4 + w2blk.size * 2 + rows * O * 4),
    )(y, ss, w2blk, b2t)

    out = out.reshape(N, H, W, O)
    return jnp.transpose(out, (0, 3, 1, 2))
